# baseline (speedup 1.0000x reference)
import functools

import jax
import jax.numpy as jnp
from jax import lax
from jax.experimental import pallas as pl
from jax.experimental.pallas import tpu as pltpu
from jax.experimental.pallas import tpu_sc as plsc

N_NODES = 10000
D_FEAT = 128
LN_EPS = 1e-5

NC = 2
NS = 16
CH = 40
RING = 5
PF = 2
LAG = RING - PF
ZROWS = 64
NPAD = 10240
AGG_DT = jnp.float32


def _sc_aggregate(x, edge_index):
    e = edge_index.shape[1]
    epw = e // (NC * NS)
    nch = epw // CH
    ngrp = nch // RING
    assert nch == ngrp * RING and epw == nch * CH
    rows_per_sub = NPAD // NS

    src1 = edge_index[0]
    dst4 = jnp.reshape(edge_index[1], (NC * NS, ngrp, RING, CH))
    mesh = plsc.VectorSubcoreMesh(core_axis_name="c", subcore_axis_name="s")

    scratch = (
        [pltpu.VMEM_SHARED((NPAD, D_FEAT), AGG_DT)]
        + [pltpu.VMEM((epw,), jnp.int32)]
        + [pltpu.VMEM((2 * RING, CH), jnp.int32)]
        + [pltpu.VMEM((CH, D_FEAT), AGG_DT)] * RING
        + [pltpu.VMEM((ZROWS, D_FEAT), AGG_DT)]
        + [pltpu.SemaphoreType.DMA] * (2 * RING + 1)
    )

    @functools.partial(
        pl.kernel,
        out_type=jax.ShapeDtypeStruct((NC * NS, NPAD // NS, D_FEAT), AGG_DT),
        mesh=mesh,
        scratch_types=scratch,
    )
    def k(x_hbm, src_hbm, dst4_hbm, out_hbm, *scr):
        aggr_sh, src_blk, dst_blk = scr[0], scr[1], scr[2]
        rowbufs = scr[3:3 + RING]
        zbuf = scr[3 + RING]
        gsems = scr[4 + RING:4 + 2 * RING]
        ssems = scr[4 + 2 * RING:4 + 3 * RING]
        isem = scr[4 + 3 * RING]

        c_ax = lax.axis_index("c")
        s_ax = lax.axis_index("s")
        w = c_ax * NS + s_ax

        def issue_gather(gg, rr, slot):
            idx = src_blk.at[pl.ds((gg * RING + rr) * CH, CH)]
            pltpu.async_copy(x_hbm.at[idx], rowbufs[slot], gsems[slot])

        def wait_gather(gg, rr, slot):
            idx = src_blk.at[pl.ds((gg * RING + rr) * CH, CH)]
            pltpu.make_async_copy(x_hbm.at[idx], rowbufs[slot],
                                  gsems[slot]).wait()

        def issue_scatter(row, r):
            pltpu.async_copy(rowbufs[r], aggr_sh.at[dst_blk.at[row]],
                             ssems[r], add=True)

        def wait_scatter(row, r):
            pltpu.make_async_copy(rowbufs[r], aggr_sh.at[dst_blk.at[row]],
                                  ssems[r]).wait()

        def issue_idx(g, poff):
            pltpu.async_copy(dst4_hbm.at[w, g],
                             dst_blk.at[pl.ds(poff, RING)], isem)

        def wait_idx(g, poff):
            pltpu.make_async_copy(dst4_hbm.at[w, g],
                                  dst_blk.at[pl.ds(poff, RING)], isem).wait()

        pltpu.sync_copy(src_hbm.at[pl.ds(w * epw, epw)], src_blk)
        pltpu.sync_copy(dst4_hbm.at[w, 0], dst_blk.at[pl.ds(0, RING)])
        issue_idx(1, RING)

        for c0 in range(PF):
            issue_gather(0, c0, c0)

        zwidth = 32 if AGG_DT == jnp.bfloat16 else 16

        @pl.loop(0, ZROWS)
        def _(i):
            @pl.loop(0, D_FEAT, step=zwidth)
            def _(j):
                zbuf[i, pl.ds(j, zwidth)] = jnp.zeros((zwidth,), AGG_DT)

        for j in range(NS):
            @pl.when(s_ax == j)
            def _():
                for i in range(rows_per_sub // ZROWS):
                    pltpu.sync_copy(
                        zbuf,
                        aggr_sh.at[pl.ds(j * rows_per_sub + i * ZROWS,
                                         ZROWS)])

        plsc.subcore_barrier()

        def step(g, r, poff, do_wait_s, do_issue_g):
            wait_gather(g, r, r)
            issue_scatter(poff + r, r)
            if do_wait_s:
                drow = (poff + r - LAG) if r >= LAG else (
                    (RING - poff) + r - LAG + RING)
                wait_scatter(drow, (r - LAG) % RING)
            if do_issue_g:
                if r + PF < RING:
                    issue_gather(g, r + PF, r + PF)
                else:
                    issue_gather(g + 1, r + PF - RING, r + PF - RING)

        for r in range(RING):
            step(0, r, 0, do_wait_s=(r >= LAG), do_issue_g=True)

        @pl.loop(0, ngrp - 2)
        def _(q):
            g = q + 1
            poff = (g % 2) * RING
            wait_idx(g, poff)
            for r in range(RING):
                step(g, r, poff, True, True)
                if r == LAG - 1:
                    issue_idx(g + 1, RING - poff)

        gl = ngrp - 1
        lpoff = (gl % 2) * RING
        wait_idx(gl, lpoff)
        for r in range(RING):
            step(gl, r, lpoff, do_wait_s=True,
                 do_issue_g=(gl * RING + r + PF < nch))
        for c in range(nch - LAG, nch):
            wait_scatter(lpoff + (c % RING), c % RING)

        plsc.subcore_barrier()

        for j in range(NS):
            @pl.when(s_ax == j)
            def _():
                pltpu.sync_copy(
                    aggr_sh.at[pl.ds(j * rows_per_sub, rows_per_sub)],
                    out_hbm.at[w])

    return k(x, src1, dst4).reshape(NC, NPAD, D_FEAT)


def _tc_update(x, partials, w, b, ln_w, ln_b):
    blk = 2000
    grid = (N_NODES // blk,)

    def body(x_ref, p_ref, w_ref, b_ref, lnw_ref, lnb_ref, o_ref):
        h = (x_ref[...] + p_ref[0].astype(jnp.float32)
             + p_ref[1].astype(jnp.float32))
        y = lax.dot_general(h, w_ref[...], (((1,), (1,)), ((), ())),
                            preferred_element_type=jnp.float32)
        y = y + b_ref[...]
        mean = jnp.mean(y, axis=-1, keepdims=True)
        var = jnp.mean((y - mean) ** 2, axis=-1, keepdims=True)
        o_ref[...] = ((y - mean) * lax.rsqrt(var + LN_EPS) * lnw_ref[...]
                      + lnb_ref[...])

    return pl.pallas_call(
        body,
        grid=grid,
        in_specs=[
            pl.BlockSpec((blk, D_FEAT), lambda i: (i, 0)),
            pl.BlockSpec((NC, blk, D_FEAT), lambda i: (0, i, 0)),
            pl.BlockSpec((D_FEAT, D_FEAT), lambda i: (0, 0)),
            pl.BlockSpec((1, D_FEAT), lambda i: (0, 0)),
            pl.BlockSpec((1, D_FEAT), lambda i: (0, 0)),
            pl.BlockSpec((1, D_FEAT), lambda i: (0, 0)),
        ],
        out_specs=pl.BlockSpec((blk, D_FEAT), lambda i: (i, 0)),
        out_shape=jax.ShapeDtypeStruct((N_NODES, D_FEAT), jnp.float32),
    )(x, partials, w, b, ln_w, ln_b)


def kernel(x, edge_index, batch, W, b, ln_w, ln_b):
    partials = _sc_aggregate(x, edge_index)
    return _tc_update(x, partials, W, b.reshape(1, D_FEAT),
                      ln_w.reshape(1, D_FEAT), ln_b.reshape(1, D_FEAT))


# speedup vs baseline: 1.2896x; 1.2896x over previous
import functools

import jax
import jax.numpy as jnp
from jax import lax
from jax.experimental import pallas as pl
from jax.experimental.pallas import tpu as pltpu
from jax.experimental.pallas import tpu_sc as plsc

N_NODES = 10000
D_FEAT = 128
LN_EPS = 1e-5

NC = 2
NS = 16
CH = 40
RING = 5
PF = 4
LAG = RING - PF
ZROWS = 64
NPAD = 10240
AGG_DT = jnp.float32


def _sc_aggregate(x, edge_index):
    e = edge_index.shape[1]
    epw = e // (NC * NS)
    nch = epw // CH
    ngrp = nch // RING
    assert nch == ngrp * RING and epw == nch * CH
    rows_per_sub = NPAD // NS

    src1 = edge_index[0]
    dst4 = jnp.reshape(edge_index[1], (NC * NS, ngrp, RING, CH))
    mesh = plsc.VectorSubcoreMesh(core_axis_name="c", subcore_axis_name="s")

    scratch = (
        [pltpu.VMEM_SHARED((NPAD, D_FEAT), AGG_DT)]
        + [pltpu.VMEM((epw,), jnp.int32)]
        + [pltpu.VMEM((2 * RING, CH), jnp.int32)]
        + [pltpu.VMEM((CH, D_FEAT), AGG_DT)] * RING
        + [pltpu.VMEM((ZROWS, D_FEAT), AGG_DT)]
        + [pltpu.SemaphoreType.DMA] * (2 * RING + 1)
    )

    @functools.partial(
        pl.kernel,
        out_type=jax.ShapeDtypeStruct((NC * NS, NPAD // NS, D_FEAT), AGG_DT),
        mesh=mesh,
        scratch_types=scratch,
    )
    def k(x_hbm, src_hbm, dst4_hbm, out_hbm, *scr):
        aggr_sh, src_blk, dst_blk = scr[0], scr[1], scr[2]
        rowbufs = scr[3:3 + RING]
        zbuf = scr[3 + RING]
        gsems = scr[4 + RING:4 + 2 * RING]
        ssems = scr[4 + 2 * RING:4 + 3 * RING]
        isem = scr[4 + 3 * RING]

        c_ax = lax.axis_index("c")
        s_ax = lax.axis_index("s")
        w = c_ax * NS + s_ax

        def issue_gather(gg, rr, slot):
            idx = src_blk.at[pl.ds((gg * RING + rr) * CH, CH)]
            pltpu.async_copy(x_hbm.at[idx], rowbufs[slot], gsems[slot])

        def wait_gather(gg, rr, slot):
            idx = src_blk.at[pl.ds((gg * RING + rr) * CH, CH)]
            pltpu.make_async_copy(x_hbm.at[idx], rowbufs[slot],
                                  gsems[slot]).wait()

        def issue_scatter(row, r):
            pltpu.async_copy(rowbufs[r], aggr_sh.at[dst_blk.at[row]],
                             ssems[r], add=True)

        def wait_scatter(row, r):
            pltpu.make_async_copy(rowbufs[r], aggr_sh.at[dst_blk.at[row]],
                                  ssems[r]).wait()

        def issue_idx(g, poff):
            pltpu.async_copy(dst4_hbm.at[w, g],
                             dst_blk.at[pl.ds(poff, RING)], isem)

        def wait_idx(g, poff):
            pltpu.make_async_copy(dst4_hbm.at[w, g],
                                  dst_blk.at[pl.ds(poff, RING)], isem).wait()

        pltpu.sync_copy(src_hbm.at[pl.ds(w * epw, epw)], src_blk)
        pltpu.sync_copy(dst4_hbm.at[w, 0], dst_blk.at[pl.ds(0, RING)])
        issue_idx(1, RING)

        for c0 in range(PF):
            issue_gather(0, c0, c0)

        zwidth = 32 if AGG_DT == jnp.bfloat16 else 16

        @pl.loop(0, ZROWS)
        def _(i):
            @pl.loop(0, D_FEAT, step=zwidth)
            def _(j):
                zbuf[i, pl.ds(j, zwidth)] = jnp.zeros((zwidth,), AGG_DT)

        for j in range(NS):
            @pl.when(s_ax == j)
            def _():
                for i in range(rows_per_sub // ZROWS):
                    pltpu.sync_copy(
                        zbuf,
                        aggr_sh.at[pl.ds(j * rows_per_sub + i * ZROWS,
                                         ZROWS)])

        plsc.subcore_barrier()

        def step(g, r, poff, do_wait_s, do_issue_g):
            wait_gather(g, r, r)
            issue_scatter(poff + r, r)
            if do_wait_s:
                drow = (poff + r - LAG) if r >= LAG else (
                    (RING - poff) + r - LAG + RING)
                wait_scatter(drow, (r - LAG) % RING)
            if do_issue_g:
                if r + PF < RING:
                    issue_gather(g, r + PF, r + PF)
                else:
                    issue_gather(g + 1, r + PF - RING, r + PF - RING)

        for r in range(RING):
            step(0, r, 0, do_wait_s=(r >= LAG), do_issue_g=True)

        @pl.loop(0, ngrp - 2)
        def _(q):
            g = q + 1
            poff = (g % 2) * RING
            wait_idx(g, poff)
            for r in range(RING):
                step(g, r, poff, True, True)
                if r == LAG - 1:
                    issue_idx(g + 1, RING - poff)

        gl = ngrp - 1
        lpoff = (gl % 2) * RING
        wait_idx(gl, lpoff)
        for r in range(RING):
            step(gl, r, lpoff, do_wait_s=True,
                 do_issue_g=(gl * RING + r + PF < nch))
        for c in range(nch - LAG, nch):
            wait_scatter(lpoff + (c % RING), c % RING)

        plsc.subcore_barrier()

        for j in range(NS):
            @pl.when(s_ax == j)
            def _():
                pltpu.sync_copy(
                    aggr_sh.at[pl.ds(j * rows_per_sub, rows_per_sub)],
                    out_hbm.at[w])

    return k(x, src1, dst4).reshape(NC, NPAD, D_FEAT)


def _tc_update(x, partials, w, b, ln_w, ln_b):
    blk = 2000
    grid = (N_NODES // blk,)

    def body(x_ref, p_ref, w_ref, b_ref, lnw_ref, lnb_ref, o_ref):
        h = (x_ref[...] + p_ref[0].astype(jnp.float32)
             + p_ref[1].astype(jnp.float32))
        y = lax.dot_general(h, w_ref[...], (((1,), (1,)), ((), ())),
                            preferred_element_type=jnp.float32)
        y = y + b_ref[...]
        mean = jnp.mean(y, axis=-1, keepdims=True)
        var = jnp.mean((y - mean) ** 2, axis=-1, keepdims=True)
        o_ref[...] = ((y - mean) * lax.rsqrt(var + LN_EPS) * lnw_ref[...]
                      + lnb_ref[...])

    return pl.pallas_call(
        body,
        grid=grid,
        in_specs=[
            pl.BlockSpec((blk, D_FEAT), lambda i: (i, 0)),
            pl.BlockSpec((NC, blk, D_FEAT), lambda i: (0, i, 0)),
            pl.BlockSpec((D_FEAT, D_FEAT), lambda i: (0, 0)),
            pl.BlockSpec((1, D_FEAT), lambda i: (0, 0)),
            pl.BlockSpec((1, D_FEAT), lambda i: (0, 0)),
            pl.BlockSpec((1, D_FEAT), lambda i: (0, 0)),
        ],
        out_specs=pl.BlockSpec((blk, D_FEAT), lambda i: (i, 0)),
        out_shape=jax.ShapeDtypeStruct((N_NODES, D_FEAT), jnp.float32),
    )(x, partials, w, b, ln_w, ln_b)


def kernel(x, edge_index, batch, W, b, ln_w, ln_b):
    partials = _sc_aggregate(x, edge_index)
    return _tc_update(x, partials, W, b.reshape(1, D_FEAT),
                      ln_w.reshape(1, D_FEAT), ln_b.reshape(1, D_FEAT))


# speedup vs baseline: 1.3201x; 1.0236x over previous
import functools

import jax
import jax.numpy as jnp
from jax import lax
from jax.experimental import pallas as pl
from jax.experimental.pallas import tpu as pltpu
from jax.experimental.pallas import tpu_sc as plsc

N_NODES = 10000
D_FEAT = 128
LN_EPS = 1e-5

NC = 2
NS = 16
CH = 40
RING = 5
PF = 4
LAG = RING - PF
ZROWS = 16
NPAD = 10240
AGG_DT = jnp.float32


def _sc_aggregate(x, edge_index):
    e = edge_index.shape[1]
    epw = e // (NC * NS)
    nch = epw // CH
    ngrp = nch // RING
    assert nch == ngrp * RING and epw == nch * CH
    rows_per_sub = NPAD // NS

    src1 = edge_index[0]
    dst1 = edge_index[1]
    mesh = plsc.VectorSubcoreMesh(core_axis_name="c", subcore_axis_name="s")

    scratch = (
        [pltpu.VMEM_SHARED((NPAD, D_FEAT), AGG_DT)]
        + [pltpu.VMEM((epw,), jnp.int32)]
        + [pltpu.VMEM((epw,), jnp.int32)]
        + [pltpu.VMEM((CH, D_FEAT), AGG_DT)] * RING
        + [pltpu.VMEM((ZROWS, D_FEAT), AGG_DT)]
        + [pltpu.SemaphoreType.DMA] * (2 * RING)
    )

    @functools.partial(
        pl.kernel,
        out_type=jax.ShapeDtypeStruct((NC * NS, NPAD // NS, D_FEAT), AGG_DT),
        mesh=mesh,
        scratch_types=scratch,
    )
    def k(x_hbm, src_hbm, dst_hbm, out_hbm, *scr):
        aggr_sh, src_blk, dst_blk = scr[0], scr[1], scr[2]
        rowbufs = scr[3:3 + RING]
        zbuf = scr[3 + RING]
        gsems = scr[4 + RING:4 + 2 * RING]
        ssems = scr[4 + 2 * RING:4 + 3 * RING]

        c_ax = lax.axis_index("c")
        s_ax = lax.axis_index("s")
        w = c_ax * NS + s_ax

        def issue_gather(gg, rr, slot):
            idx = src_blk.at[pl.ds((gg * RING + rr) * CH, CH)]
            pltpu.async_copy(x_hbm.at[idx], rowbufs[slot], gsems[slot])

        def wait_gather(gg, rr, slot):
            idx = src_blk.at[pl.ds((gg * RING + rr) * CH, CH)]
            pltpu.make_async_copy(x_hbm.at[idx], rowbufs[slot],
                                  gsems[slot]).wait()

        def issue_scatter(gg, rr, r):
            idx = dst_blk.at[pl.ds((gg * RING + rr) * CH, CH)]
            pltpu.async_copy(rowbufs[r], aggr_sh.at[idx], ssems[r], add=True)

        def wait_scatter(gg, rr, r):
            idx = dst_blk.at[pl.ds((gg * RING + rr) * CH, CH)]
            pltpu.make_async_copy(rowbufs[r], aggr_sh.at[idx],
                                  ssems[r]).wait()

        pltpu.sync_copy(src_hbm.at[pl.ds(w * epw, epw)], src_blk)
        pltpu.sync_copy(dst_hbm.at[pl.ds(w * epw, epw)], dst_blk)

        for c0 in range(PF):
            issue_gather(0, c0, c0)

        zwidth = 32 if AGG_DT == jnp.bfloat16 else 16

        @pl.loop(0, ZROWS)
        def _(i):
            @pl.loop(0, D_FEAT, step=zwidth)
            def _(j):
                zbuf[i, pl.ds(j, zwidth)] = jnp.zeros((zwidth,), AGG_DT)

        for j in range(NS):
            @pl.when(s_ax == j)
            def _():
                for i in range(rows_per_sub // ZROWS):
                    pltpu.sync_copy(
                        zbuf,
                        aggr_sh.at[pl.ds(j * rows_per_sub + i * ZROWS,
                                         ZROWS)])

        plsc.subcore_barrier()

        def step(g, r, do_wait_s, do_issue_g):
            wait_gather(g, r, r)
            issue_scatter(g, r, r)
            if do_wait_s:
                if r >= LAG:
                    wait_scatter(g, r - LAG, r - LAG)
                else:
                    wait_scatter(g - 1, r - LAG + RING, r - LAG + RING)
            if do_issue_g:
                if r + PF < RING:
                    issue_gather(g, r + PF, r + PF)
                else:
                    issue_gather(g + 1, r + PF - RING, r + PF - RING)

        for r in range(RING):
            step(0, r, do_wait_s=(r >= LAG), do_issue_g=True)

        @pl.loop(0, ngrp - 2)
        def _(q):
            for r in range(RING):
                step(q + 1, r, True, True)

        gl = ngrp - 1
        for r in range(RING):
            step(gl, r, do_wait_s=True,
                 do_issue_g=(gl * RING + r + PF < nch))
        for c in range(nch - LAG, nch):
            wait_scatter(gl, c % RING, c % RING)

        plsc.subcore_barrier()

        for j in range(NS):
            @pl.when(s_ax == j)
            def _():
                pltpu.sync_copy(
                    aggr_sh.at[pl.ds(j * rows_per_sub, rows_per_sub)],
                    out_hbm.at[w])

    return k(x, src1, dst1).reshape(NC, NPAD, D_FEAT)


def _tc_update(x, partials, w, b, ln_w, ln_b):
    blk = 2000
    grid = (N_NODES // blk,)

    def body(x_ref, p_ref, w_ref, b_ref, lnw_ref, lnb_ref, o_ref):
        h = (x_ref[...] + p_ref[0].astype(jnp.float32)
             + p_ref[1].astype(jnp.float32))
        y = lax.dot_general(h, w_ref[...], (((1,), (1,)), ((), ())),
                            preferred_element_type=jnp.float32)
        y = y + b_ref[...]
        mean = jnp.mean(y, axis=-1, keepdims=True)
        var = jnp.mean((y - mean) ** 2, axis=-1, keepdims=True)
        o_ref[...] = ((y - mean) * lax.rsqrt(var + LN_EPS) * lnw_ref[...]
                      + lnb_ref[...])

    return pl.pallas_call(
        body,
        grid=grid,
        in_specs=[
            pl.BlockSpec((blk, D_FEAT), lambda i: (i, 0)),
            pl.BlockSpec((NC, blk, D_FEAT), lambda i: (0, i, 0)),
            pl.BlockSpec((D_FEAT, D_FEAT), lambda i: (0, 0)),
            pl.BlockSpec((1, D_FEAT), lambda i: (0, 0)),
            pl.BlockSpec((1, D_FEAT), lambda i: (0, 0)),
            pl.BlockSpec((1, D_FEAT), lambda i: (0, 0)),
        ],
        out_specs=pl.BlockSpec((blk, D_FEAT), lambda i: (i, 0)),
        out_shape=jax.ShapeDtypeStruct((N_NODES, D_FEAT), jnp.float32),
    )(x, partials, w, b, ln_w, ln_b)


def kernel(x, edge_index, batch, W, b, ln_w, ln_b):
    partials = _sc_aggregate(x, edge_index)
    return _tc_update(x, partials, W, b.reshape(1, D_FEAT),
                      ln_w.reshape(1, D_FEAT), ln_b.reshape(1, D_FEAT))


# speedup vs baseline: 1.3626x; 1.0322x over previous
import functools

import jax
import jax.numpy as jnp
from jax import lax
from jax.experimental import pallas as pl
from jax.experimental.pallas import tpu as pltpu
from jax.experimental.pallas import tpu_sc as plsc

N_NODES = 10000
D_FEAT = 128
LN_EPS = 1e-5

NC = 2
NS = 16
CH = 40
RING = 5
PF = 4
LAG = RING - PF
ZROWS = 16
NPAD = 10240
AGG_DT = jnp.float32


def _sc_aggregate(x, edge_index):
    e = edge_index.shape[1]
    epw = e // (NC * NS)
    nch = epw // CH
    ngrp = nch // RING
    assert nch == ngrp * RING and epw == nch * CH
    rows_per_sub = NPAD // NS

    src1 = edge_index[0]
    dst1 = edge_index[1]
    mesh = plsc.VectorSubcoreMesh(core_axis_name="c", subcore_axis_name="s")

    scratch = (
        [pltpu.VMEM_SHARED((NPAD, D_FEAT), AGG_DT)]
        + [pltpu.VMEM((epw,), jnp.int32)]
        + [pltpu.VMEM((epw,), jnp.int32)]
        + [pltpu.VMEM((CH, D_FEAT), AGG_DT)] * RING
        + [pltpu.VMEM((ZROWS, D_FEAT), AGG_DT)]
        + [pltpu.SemaphoreType.DMA] * (2 * RING + 1)
    )

    @functools.partial(
        pl.kernel,
        out_type=jax.ShapeDtypeStruct((NC * NS, NPAD // NS, D_FEAT), AGG_DT),
        mesh=mesh,
        scratch_types=scratch,
    )
    def k(x_hbm, src_hbm, dst_hbm, out_hbm, *scr):
        aggr_sh, src_blk, dst_blk = scr[0], scr[1], scr[2]
        rowbufs = scr[3:3 + RING]
        zbuf = scr[3 + RING]
        gsems = scr[4 + RING:4 + 2 * RING]
        ssems = scr[4 + 2 * RING:4 + 3 * RING]
        zsem = scr[4 + 3 * RING]

        c_ax = lax.axis_index("c")
        s_ax = lax.axis_index("s")
        w = c_ax * NS + s_ax

        def issue_gather(gg, rr, slot):
            idx = src_blk.at[pl.ds((gg * RING + rr) * CH, CH)]
            pltpu.async_copy(x_hbm.at[idx], rowbufs[slot], gsems[slot])

        def wait_gather(gg, rr, slot):
            idx = src_blk.at[pl.ds((gg * RING + rr) * CH, CH)]
            pltpu.make_async_copy(x_hbm.at[idx], rowbufs[slot],
                                  gsems[slot]).wait()

        def issue_scatter(gg, rr, r):
            idx = dst_blk.at[pl.ds((gg * RING + rr) * CH, CH)]
            pltpu.async_copy(rowbufs[r], aggr_sh.at[idx], ssems[r], add=True)

        def wait_scatter(gg, rr, r):
            idx = dst_blk.at[pl.ds((gg * RING + rr) * CH, CH)]
            pltpu.make_async_copy(rowbufs[r], aggr_sh.at[idx],
                                  ssems[r]).wait()

        pltpu.sync_copy(src_hbm.at[pl.ds(w * epw, epw)], src_blk)
        pltpu.sync_copy(dst_hbm.at[pl.ds(w * epw, epw)], dst_blk)

        for c0 in range(PF):
            issue_gather(0, c0, c0)

        zwidth = 32 if AGG_DT == jnp.bfloat16 else 16

        @pl.loop(0, ZROWS)
        def _(i):
            @pl.loop(0, D_FEAT, step=zwidth)
            def _(j):
                zbuf[i, pl.ds(j, zwidth)] = jnp.zeros((zwidth,), AGG_DT)

        def zslice(i):
            return aggr_sh.at[pl.ds(s_ax * rows_per_sub + i * ZROWS, ZROWS)]

        @pl.loop(0, rows_per_sub // ZROWS)
        def _(i):
            pltpu.async_copy(zbuf, zslice(i), zsem)

        @pl.loop(0, rows_per_sub // ZROWS)
        def _(i):
            pltpu.make_async_copy(zbuf, zslice(i), zsem).wait()

        plsc.subcore_barrier()

        def step(g, r, do_wait_s, do_issue_g):
            wait_gather(g, r, r)
            issue_scatter(g, r, r)
            if do_wait_s:
                if r >= LAG:
                    wait_scatter(g, r - LAG, r - LAG)
                else:
                    wait_scatter(g - 1, r - LAG + RING, r - LAG + RING)
            if do_issue_g:
                if r + PF < RING:
                    issue_gather(g, r + PF, r + PF)
                else:
                    issue_gather(g + 1, r + PF - RING, r + PF - RING)

        for r in range(RING):
            step(0, r, do_wait_s=(r >= LAG), do_issue_g=True)

        @pl.loop(0, ngrp - 2)
        def _(q):
            for r in range(RING):
                step(q + 1, r, True, True)

        gl = ngrp - 1
        for r in range(RING):
            step(gl, r, do_wait_s=True,
                 do_issue_g=(gl * RING + r + PF < nch))
        for c in range(nch - LAG, nch):
            wait_scatter(gl, c % RING, c % RING)

        plsc.subcore_barrier()

        for j in range(NS):
            @pl.when(s_ax == j)
            def _():
                pltpu.sync_copy(
                    aggr_sh.at[pl.ds(j * rows_per_sub, rows_per_sub)],
                    out_hbm.at[w])

    return k(x, src1, dst1).reshape(NC, NPAD, D_FEAT)


def _tc_update(x, partials, w, b, ln_w, ln_b):
    blk = 2000
    grid = (N_NODES // blk,)

    def body(x_ref, p_ref, w_ref, b_ref, lnw_ref, lnb_ref, o_ref):
        h = (x_ref[...] + p_ref[0].astype(jnp.float32)
             + p_ref[1].astype(jnp.float32))
        y = lax.dot_general(h, w_ref[...], (((1,), (1,)), ((), ())),
                            preferred_element_type=jnp.float32)
        y = y + b_ref[...]
        mean = jnp.mean(y, axis=-1, keepdims=True)
        var = jnp.mean((y - mean) ** 2, axis=-1, keepdims=True)
        o_ref[...] = ((y - mean) * lax.rsqrt(var + LN_EPS) * lnw_ref[...]
                      + lnb_ref[...])

    return pl.pallas_call(
        body,
        grid=grid,
        in_specs=[
            pl.BlockSpec((blk, D_FEAT), lambda i: (i, 0)),
            pl.BlockSpec((NC, blk, D_FEAT), lambda i: (0, i, 0)),
            pl.BlockSpec((D_FEAT, D_FEAT), lambda i: (0, 0)),
            pl.BlockSpec((1, D_FEAT), lambda i: (0, 0)),
            pl.BlockSpec((1, D_FEAT), lambda i: (0, 0)),
            pl.BlockSpec((1, D_FEAT), lambda i: (0, 0)),
        ],
        out_specs=pl.BlockSpec((blk, D_FEAT), lambda i: (i, 0)),
        out_shape=jax.ShapeDtypeStruct((N_NODES, D_FEAT), jnp.float32),
    )(x, partials, w, b, ln_w, ln_b)


def kernel(x, edge_index, batch, W, b, ln_w, ln_b):
    partials = _sc_aggregate(x, edge_index)
    return _tc_update(x, partials, W, b.reshape(1, D_FEAT),
                      ln_w.reshape(1, D_FEAT), ln_b.reshape(1, D_FEAT))
